# baseline (speedup 1.0000x reference)
import functools

import jax
import jax.numpy as jnp
from jax import lax
from jax.experimental import pallas as pl
from jax.experimental.pallas import tpu as pltpu
from jax.experimental.pallas import tpu_sc as plsc

N_NODES = 10000
N_EDGES = 160000

NC = 2
NT = 16
W = 128
K = 128
HALVES = 2
CHUNKS_H = 40
EPT_PAD = HALVES * CHUNKS_H * K
PAD = EPT_PAD - N_EDGES // NT
ACC_ROWS = 10240
STRIPE = ACC_ROWS // NT
LAST = N_NODES - 15 * STRIPE

_MESH = plsc.VectorSubcoreMesh(core_axis_name="c", subcore_axis_name="s")


def _make_agg(n_slices, with_deg):
    n_passes = n_slices // NC
    n_out = n_slices + (2 if with_deg else 0)

    def body(src_hbm, dst_hbm, zeros_hbm, *rest):
        ys = rest[:n_slices]
        outs = rest[n_slices:n_slices + n_out]
        (acc, sidx, didx, gbuf0, gbuf1,
         sem0, sem1, sem2, sem3) = rest[n_slices + n_out:]
        c = lax.axis_index("c")
        s = lax.axis_index("s")
        row0 = s * STRIPE

        def copyout(out):
            @pl.when(s < NT - 1)
            def _():
                pltpu.sync_copy(acc.at[pl.ds(row0, STRIPE)],
                                out.at[pl.ds(row0, STRIPE)])
            @pl.when(s == NT - 1)
            def _():
                pltpu.sync_copy(acc.at[pl.ds((NT - 1) * STRIPE, LAST)],
                                out.at[pl.ds((NT - 1) * STRIPE, LAST)])

        def accumulate_edges(y_core, halves=tuple(range(HALVES))):
            for half in halves:
                pltpu.sync_copy(dst_hbm.at[s, half], didx)
                if y_core is None:
                    def chunk(j, _):
                        pltpu.sync_copy(gbuf0, acc.at[didx.at[j]], add=True)
                        return 0
                    lax.fori_loop(0, CHUNKS_H, chunk, 0)
                else:
                    pltpu.sync_copy(src_hbm.at[s, half], sidx)
                    npair = CHUNKS_H // 2
                    pltpu.async_copy(y_core.at[sidx.at[0]], gbuf0, sem0)
                    pltpu.async_copy(y_core.at[sidx.at[1]], gbuf1, sem1)

                    def pair(m, _):
                        j0 = 2 * m
                        pltpu.make_async_copy(
                            y_core.at[sidx.at[j0]], gbuf0, sem0).wait()
                        s0 = pltpu.async_copy(
                            gbuf0, acc.at[didx.at[j0]], sem2, add=True)
                        pltpu.make_async_copy(
                            y_core.at[sidx.at[j0 + 1]], gbuf1, sem1).wait()
                        s1 = pltpu.async_copy(
                            gbuf1, acc.at[didx.at[j0 + 1]], sem3, add=True)
                        s0.wait()

                        @pl.when(m < npair - 1)
                        def _():
                            pltpu.async_copy(
                                y_core.at[sidx.at[j0 + 2]], gbuf0, sem0)
                        s1.wait()

                        @pl.when(m < npair - 1)
                        def _():
                            pltpu.async_copy(
                                y_core.at[sidx.at[j0 + 3]], gbuf1, sem1)
                        return 0
                    lax.fori_loop(0, npair, pair, 0)

        def scatter_pass(y0, y1, out0, out1):
            pltpu.sync_copy(zeros_hbm, acc.at[pl.ds(row0, STRIPE)])
            if y0 is None:
                def fill(i, _):
                    for cc in range(W // 16):
                        gbuf0[i, pl.ds(cc * 16, 16)] = jnp.full(
                            (16,), 1.0, jnp.float32)
                    return 0
                lax.fori_loop(0, K, fill, 0)
            plsc.subcore_barrier()

            if y0 is None:
                @pl.when(c == 0)
                def _():
                    accumulate_edges(None, (0,))
                @pl.when(c == 1)
                def _():
                    accumulate_edges(None, (1,))
            else:
                @pl.when(c == 0)
                def _():
                    accumulate_edges(y0)
                @pl.when(c == 1)
                def _():
                    accumulate_edges(y1)
            plsc.subcore_barrier()

            @pl.when(c == 0)
            def _():
                copyout(out0)
            @pl.when(c == 1)
            def _():
                copyout(out1)

        for p in range(n_passes):
            scatter_pass(ys[NC * p], ys[NC * p + 1],
                         outs[NC * p], outs[NC * p + 1])
        if with_deg:
            scatter_pass(None, None, outs[n_slices], outs[n_slices + 1])

    out_type = [jax.ShapeDtypeStruct((N_NODES, W), jnp.float32)] * n_out
    scratch = [
        pltpu.VMEM_SHARED((ACC_ROWS, W), jnp.float32),
        pltpu.VMEM((CHUNKS_H, K), jnp.int32),
        pltpu.VMEM((CHUNKS_H, K), jnp.int32),
        pltpu.VMEM((K, W), jnp.float32),
        pltpu.VMEM((K, W), jnp.float32),
        pltpu.SemaphoreType.DMA,
        pltpu.SemaphoreType.DMA,
        pltpu.SemaphoreType.DMA,
        pltpu.SemaphoreType.DMA,
    ]
    return pl.kernel(body, out_type=out_type, mesh=_MESH, scratch_types=scratch)


BN = 1000


def _layer_body(n_in, n_out, relu, *refs):
    aggs = refs[:n_in]
    ys = refs[n_in:2 * n_in]
    wl_ref, wr_ref, b_ref, dega_ref, degb_ref = refs[2 * n_in:2 * n_in + 5]
    o_refs = refs[2 * n_in + 5:]
    agg = jnp.concatenate([r[...] for r in aggs], axis=1)
    y = jnp.concatenate([r[...] for r in ys], axis=1)
    deg = dega_ref[:, 0:1] + degb_ref[:, 0:1]
    inv = 1.0 / jnp.maximum(deg, 1.0)
    acc = jnp.dot(agg * inv, wl_ref[...], preferred_element_type=jnp.float32)
    acc += jnp.dot(y, wr_ref[...], preferred_element_type=jnp.float32)
    acc += b_ref[...]
    if relu:
        acc = jnp.maximum(acc, 0.0)
    for i in range(n_out):
        o_refs[i][...] = acc[:, i * W:(i + 1) * W]


def _tc_layer(aggs, ys, wl, wr, b, deg, relu):
    n_in = len(aggs)
    din = n_in * W
    dout = wl.shape[1]
    n_out = dout // W
    sl = pl.BlockSpec((BN, W), lambda i: (i, 0))
    return pl.pallas_call(
        functools.partial(_layer_body, n_in, n_out, relu),
        grid=(N_NODES // BN,),
        in_specs=[sl] * (2 * n_in) + [
            pl.BlockSpec((din, dout), lambda i: (0, 0)),
            pl.BlockSpec((din, dout), lambda i: (0, 0)),
            pl.BlockSpec((1, dout), lambda i: (0, 0)),
            sl,
            sl,
        ],
        out_specs=[sl] * n_out,
        out_shape=[jax.ShapeDtypeStruct((N_NODES, W), jnp.float32)] * n_out,
    )(*aggs, *ys, wl, wr, b.reshape(1, dout), deg[0], deg[1])


def _mm_body(n_in, n_out, *refs):
    ys = refs[:n_in]
    w_ref, b_ref = refs[n_in:n_in + 2]
    o_refs = refs[n_in + 2:]
    y = jnp.concatenate([r[...] for r in ys], axis=1)
    acc = jnp.dot(y, w_ref[...], preferred_element_type=jnp.float32) + b_ref[...]
    if n_out == 1:
        o_refs[0][...] = acc
    else:
        for i in range(n_out):
            o_refs[i][...] = acc[:, i * W:(i + 1) * W]


def _tc_mm(ys, w, b, slice_out=True):
    n_in = len(ys)
    din = n_in * W
    dout = w.shape[1]
    n_out = dout // W if slice_out else 1
    ow = W if slice_out else dout
    sl = pl.BlockSpec((BN, W), lambda i: (i, 0))
    osl = pl.BlockSpec((BN, ow), lambda i: (i, 0))
    return pl.pallas_call(
        functools.partial(_mm_body, n_in, n_out),
        grid=(N_NODES // BN,),
        in_specs=[sl] * n_in + [
            pl.BlockSpec((din, dout), lambda i: (0, 0)),
            pl.BlockSpec((1, dout), lambda i: (0, 0)),
        ],
        out_specs=[osl] * n_out,
        out_shape=[jax.ShapeDtypeStruct((N_NODES, ow), jnp.float32)] * n_out,
    )(*ys, w, b.reshape(1, dout))


def _final_body(n_in, *refs):
    aggs = refs[:n_in]
    dega_ref, degb_ref, r_ref = refs[n_in:n_in + 3]
    o_ref = refs[n_in + 3]
    agg = jnp.concatenate([r[...] for r in aggs], axis=1)
    deg = dega_ref[:, 0:1] + degb_ref[:, 0:1]
    inv = 1.0 / jnp.maximum(deg, 1.0)
    o_ref[...] = agg * inv + r_ref[...]


def _tc_final(aggs, deg, r3):
    n_in = len(aggs)
    dout = n_in * W
    sl = pl.BlockSpec((BN, W), lambda i: (i, 0))
    return pl.pallas_call(
        functools.partial(_final_body, n_in),
        grid=(N_NODES // BN,),
        in_specs=[sl] * n_in + [sl, sl,
                                pl.BlockSpec((BN, dout), lambda i: (i, 0))],
        out_specs=pl.BlockSpec((BN, dout), lambda i: (i, 0)),
        out_shape=jax.ShapeDtypeStruct((N_NODES, dout), jnp.float32),
    )(*aggs, deg[0], deg[1], r3)


def kernel(x, edge_index, Wl1, Wr1, b1, Wl2, Wr2, b2, Wl3, Wr3, b3):
    src = edge_index[0].astype(jnp.int32).reshape(NT, N_EDGES // NT)
    dst = edge_index[1].astype(jnp.int32).reshape(NT, N_EDGES // NT)
    src = jnp.concatenate([src, jnp.zeros((NT, PAD), jnp.int32)], axis=1)
    dst = jnp.concatenate(
        [dst, jnp.full((NT, PAD), N_NODES, jnp.int32)], axis=1)
    src = src.reshape(NT, HALVES, CHUNKS_H, K)
    dst = dst.reshape(NT, HALVES, CHUNKS_H, K)
    zeros = jnp.zeros((STRIPE, W), jnp.float32)

    xs = [lax.slice(x, (0, i * W), (N_NODES, (i + 1) * W)) for i in range(2)]

    agg1_0, agg1_1, deg_a, deg_b = _make_agg(2, True)(src, dst, zeros, *xs)
    deg = (deg_a, deg_b)
    h1s = _tc_layer([agg1_0, agg1_1], xs, Wl1, Wr1, b1, deg, relu=True)

    agg2s = _make_agg(8, False)(src, dst, zeros, *h1s)
    h2s = _tc_layer(list(agg2s), list(h1s), Wl2, Wr2, b2, deg, relu=True)

    ts = _tc_mm(list(h2s), Wl3, jnp.zeros((512,), jnp.float32))
    agg3s = _make_agg(4, False)(src, dst, zeros, *ts)
    (r3,) = _tc_mm(list(h2s), Wr3, b3, slice_out=False)
    return _tc_final(list(agg3s), deg, r3)


# speedup vs baseline: 1.0401x; 1.0401x over previous
import functools

import jax
import jax.numpy as jnp
from jax import lax
from jax.experimental import pallas as pl
from jax.experimental.pallas import tpu as pltpu
from jax.experimental.pallas import tpu_sc as plsc

N_NODES = 10000
N_EDGES = 160000

NC = 2
NT = 16
W = 128
K = 128
HALVES = 2
CHUNKS_H = 40
EPT_PAD = HALVES * CHUNKS_H * K
PAD = EPT_PAD - N_EDGES // NT
ACC_ROWS = 10240
STRIPE = ACC_ROWS // NT
LAST = N_NODES - 15 * STRIPE

_MESH = plsc.VectorSubcoreMesh(core_axis_name="c", subcore_axis_name="s")


def _make_agg(n_slices, with_deg):
    n_passes = n_slices // NC
    n_out = n_slices + (1 if with_deg else 0)

    def body(src_hbm, dst_hbm, zeros_hbm, *rest):
        ys = rest[:n_slices]
        outs = rest[n_slices:n_slices + n_out]
        (acc, sidx, didx, gbuf0, gbuf1,
         sem0, sem1, sem2, sem3) = rest[n_slices + n_out:]
        c = lax.axis_index("c")
        s = lax.axis_index("s")
        row0 = s * STRIPE

        def copyout(out):
            @pl.when(s < NT - 1)
            def _():
                pltpu.sync_copy(acc.at[pl.ds(row0, STRIPE)],
                                out.at[pl.ds(row0, STRIPE)])
            @pl.when(s == NT - 1)
            def _():
                pltpu.sync_copy(acc.at[pl.ds((NT - 1) * STRIPE, LAST)],
                                out.at[pl.ds((NT - 1) * STRIPE, LAST)])

        def accumulate_edges(y_core):
            for half in range(HALVES):
                pltpu.sync_copy(dst_hbm.at[s, half], didx)
                if y_core is None:
                    def chunk(j, _):
                        pltpu.sync_copy(gbuf0, acc.at[didx.at[j]], add=True)
                        return 0
                    lax.fori_loop(0, CHUNKS_H, chunk, 0)
                else:
                    pltpu.sync_copy(src_hbm.at[s, half], sidx)
                    npair = CHUNKS_H // 2
                    pltpu.async_copy(y_core.at[sidx.at[0]], gbuf0, sem0)
                    pltpu.async_copy(y_core.at[sidx.at[1]], gbuf1, sem1)

                    def pair(m, _):
                        j0 = 2 * m
                        pltpu.make_async_copy(
                            y_core.at[sidx.at[j0]], gbuf0, sem0).wait()
                        s0 = pltpu.async_copy(
                            gbuf0, acc.at[didx.at[j0]], sem2, add=True)
                        pltpu.make_async_copy(
                            y_core.at[sidx.at[j0 + 1]], gbuf1, sem1).wait()
                        s1 = pltpu.async_copy(
                            gbuf1, acc.at[didx.at[j0 + 1]], sem3, add=True)
                        s0.wait()

                        @pl.when(m < npair - 1)
                        def _():
                            pltpu.async_copy(
                                y_core.at[sidx.at[j0 + 2]], gbuf0, sem0)
                        s1.wait()

                        @pl.when(m < npair - 1)
                        def _():
                            pltpu.async_copy(
                                y_core.at[sidx.at[j0 + 3]], gbuf1, sem1)
                        return 0
                    lax.fori_loop(0, npair, pair, 0)

        def scatter_pass(y0, y1, out0, out1):
            pltpu.sync_copy(zeros_hbm, acc.at[pl.ds(row0, STRIPE)])
            if y0 is None:
                def fill(i, _):
                    for cc in range(W // 16):
                        gbuf0[i, pl.ds(cc * 16, 16)] = jnp.full(
                            (16,), 1.0, jnp.float32)
                    return 0
                lax.fori_loop(0, K, fill, 0)
            plsc.subcore_barrier()

            @pl.when(c == 0)
            def _():
                accumulate_edges(y0)
            if y1 is not None or out1 is not None:
                @pl.when(c == 1)
                def _():
                    accumulate_edges(y1)
            plsc.subcore_barrier()

            @pl.when(c == 0)
            def _():
                copyout(out0)
            if out1 is not None:
                @pl.when(c == 1)
                def _():
                    copyout(out1)

        for p in range(n_passes):
            scatter_pass(ys[NC * p], ys[NC * p + 1],
                         outs[NC * p], outs[NC * p + 1])
        if with_deg:
            scatter_pass(None, None, outs[n_slices], None)

    out_type = [jax.ShapeDtypeStruct((N_NODES, W), jnp.float32)] * n_out
    scratch = [
        pltpu.VMEM_SHARED((ACC_ROWS, W), jnp.float32),
        pltpu.VMEM((CHUNKS_H, K), jnp.int32),
        pltpu.VMEM((CHUNKS_H, K), jnp.int32),
        pltpu.VMEM((K, W), jnp.float32),
        pltpu.VMEM((K, W), jnp.float32),
        pltpu.SemaphoreType.DMA,
        pltpu.SemaphoreType.DMA,
        pltpu.SemaphoreType.DMA,
        pltpu.SemaphoreType.DMA,
    ]
    return pl.kernel(body, out_type=out_type, mesh=_MESH, scratch_types=scratch)


BN = 1000


def _layer_body(n_in, n_out, relu, *refs):
    aggs = refs[:n_in]
    wl_ref, r_ref, deg_ref = refs[n_in:n_in + 3]
    o_refs = refs[n_in + 3:]
    agg = jnp.concatenate([r[...] for r in aggs], axis=1)
    inv = 1.0 / jnp.maximum(deg_ref[:, 0:1], 1.0)
    acc = jnp.dot(agg * inv, wl_ref[...], preferred_element_type=jnp.float32)
    acc += r_ref[...]
    if relu:
        acc = jnp.maximum(acc, 0.0)
    for i in range(n_out):
        o_refs[i][...] = acc[:, i * W:(i + 1) * W]


def _tc_layer(aggs, wl, r, deg, relu):
    n_in = len(aggs)
    din = n_in * W
    dout = wl.shape[1]
    n_out = dout // W
    sl = pl.BlockSpec((BN, W), lambda i: (i, 0))
    return pl.pallas_call(
        functools.partial(_layer_body, n_in, n_out, relu),
        grid=(N_NODES // BN,),
        in_specs=[sl] * n_in + [
            pl.BlockSpec((din, dout), lambda i: (0, 0)),
            pl.BlockSpec((BN, dout), lambda i: (i, 0)),
            sl,
        ],
        out_specs=[sl] * n_out,
        out_shape=[jax.ShapeDtypeStruct((N_NODES, W), jnp.float32)] * n_out,
    )(*aggs, wl, r, deg)


def _mm_body(n_in, n_out, *refs):
    ys = refs[:n_in]
    w_ref, b_ref = refs[n_in:n_in + 2]
    o_refs = refs[n_in + 2:]
    y = jnp.concatenate([r[...] for r in ys], axis=1)
    acc = jnp.dot(y, w_ref[...], preferred_element_type=jnp.float32) + b_ref[...]
    if n_out == 1:
        o_refs[0][...] = acc
    else:
        for i in range(n_out):
            o_refs[i][...] = acc[:, i * W:(i + 1) * W]


def _tc_mm(ys, w, b, slice_out=True):
    n_in = len(ys)
    din = n_in * W
    dout = w.shape[1]
    n_out = dout // W if slice_out else 1
    ow = W if slice_out else dout
    sl = pl.BlockSpec((BN, W), lambda i: (i, 0))
    osl = pl.BlockSpec((BN, ow), lambda i: (i, 0))
    return pl.pallas_call(
        functools.partial(_mm_body, n_in, n_out),
        grid=(N_NODES // BN,),
        in_specs=[sl] * n_in + [
            pl.BlockSpec((din, dout), lambda i: (0, 0)),
            pl.BlockSpec((1, dout), lambda i: (0, 0)),
        ],
        out_specs=[osl] * n_out,
        out_shape=[jax.ShapeDtypeStruct((N_NODES, ow), jnp.float32)] * n_out,
    )(*ys, w, b.reshape(1, dout))


def _final_body(n_in, *refs):
    aggs = refs[:n_in]
    deg_ref, r_ref = refs[n_in:n_in + 2]
    o_ref = refs[n_in + 2]
    agg = jnp.concatenate([r[...] for r in aggs], axis=1)
    inv = 1.0 / jnp.maximum(deg_ref[:, 0:1], 1.0)
    o_ref[...] = agg * inv + r_ref[...]


def _tc_final(aggs, deg, r3):
    n_in = len(aggs)
    dout = n_in * W
    sl = pl.BlockSpec((BN, W), lambda i: (i, 0))
    return pl.pallas_call(
        functools.partial(_final_body, n_in),
        grid=(N_NODES // BN,),
        in_specs=[sl] * n_in + [sl, pl.BlockSpec((BN, dout), lambda i: (i, 0))],
        out_specs=pl.BlockSpec((BN, dout), lambda i: (i, 0)),
        out_shape=jax.ShapeDtypeStruct((N_NODES, dout), jnp.float32),
    )(*aggs, deg, r3)


def kernel(x, edge_index, Wl1, Wr1, b1, Wl2, Wr2, b2, Wl3, Wr3, b3):
    src = edge_index[0].astype(jnp.int32).reshape(NT, N_EDGES // NT)
    dst = edge_index[1].astype(jnp.int32).reshape(NT, N_EDGES // NT)
    src = jnp.concatenate([src, jnp.zeros((NT, PAD), jnp.int32)], axis=1)
    dst = jnp.concatenate(
        [dst, jnp.full((NT, PAD), N_NODES, jnp.int32)], axis=1)
    src = src.reshape(NT, HALVES, CHUNKS_H, K)
    dst = dst.reshape(NT, HALVES, CHUNKS_H, K)
    zeros = jnp.zeros((STRIPE, W), jnp.float32)

    xs = [lax.slice(x, (0, i * W), (N_NODES, (i + 1) * W)) for i in range(2)]

    agg1 = _make_agg(2, True)(src, dst, zeros, *xs)
    (r1,) = _tc_mm(xs, Wr1, b1, slice_out=False)
    agg1_0, agg1_1, deg = agg1
    h1s = _tc_layer([agg1_0, agg1_1], Wl1, r1, deg, relu=True)

    agg2s = _make_agg(8, False)(src, dst, zeros, *h1s)
    (r2,) = _tc_mm(list(h1s), Wr2, b2, slice_out=False)
    h2s = _tc_layer(list(agg2s), Wl2, r2, deg, relu=True)

    ts = _tc_mm(list(h2s), Wl3, jnp.zeros((512,), jnp.float32))
    agg3s = _make_agg(4, False)(src, dst, zeros, *ts)
    (r3,) = _tc_mm(list(h2s), Wr3, b3, slice_out=False)
    return _tc_final(list(agg3s), deg, r3)


# speedup vs baseline: 1.0477x; 1.0073x over previous
import functools

import jax
import jax.numpy as jnp
from jax import lax
from jax.experimental import pallas as pl
from jax.experimental.pallas import tpu as pltpu
from jax.experimental.pallas import tpu_sc as plsc

N_NODES = 10000
N_EDGES = 160000

NC = 2
NT = 16
W = 128
K = 128
HALVES = 2
CHUNKS_H = 40
EPT_PAD = HALVES * CHUNKS_H * K
PAD = EPT_PAD - N_EDGES // NT
ACC_ROWS = 10240
STRIPE = ACC_ROWS // NT
LAST = N_NODES - 15 * STRIPE

_MESH = plsc.VectorSubcoreMesh(core_axis_name="c", subcore_axis_name="s")


def _make_agg(n_slices, with_deg):
    n_passes = n_slices // NC
    n_out = n_slices + (1 if with_deg else 0)

    def body(src_hbm, dst_hbm, zeros_hbm, *rest):
        ys = rest[:n_slices]
        outs = rest[n_slices:n_slices + n_out]
        (acc, sidx, didx, gbuf0, gbuf1,
         sem0, sem1, sem2, sem3) = rest[n_slices + n_out:]
        c = lax.axis_index("c")
        s = lax.axis_index("s")
        row0 = s * STRIPE

        def copyout(out):
            @pl.when(s < NT - 1)
            def _():
                pltpu.sync_copy(acc.at[pl.ds(row0, STRIPE)],
                                out.at[pl.ds(row0, STRIPE)])
            @pl.when(s == NT - 1)
            def _():
                pltpu.sync_copy(acc.at[pl.ds((NT - 1) * STRIPE, LAST)],
                                out.at[pl.ds((NT - 1) * STRIPE, LAST)])

        def accumulate_edges(y_core):
            for half in range(HALVES):
                pltpu.sync_copy(dst_hbm.at[s, half], didx)
                if y_core is None:
                    def chunk(j, _):
                        pltpu.sync_copy(gbuf0, acc.at[didx.at[j]], add=True)
                        return 0
                    lax.fori_loop(0, CHUNKS_H, chunk, 0)
                else:
                    pltpu.sync_copy(src_hbm.at[s, half], sidx)
                    npair = CHUNKS_H // 2
                    pltpu.async_copy(y_core.at[sidx.at[0]], gbuf0, sem0)
                    pltpu.async_copy(y_core.at[sidx.at[1]], gbuf1, sem1)

                    def pair(m, _):
                        j0 = 2 * m
                        pltpu.make_async_copy(
                            y_core.at[sidx.at[j0]], gbuf0, sem0).wait()
                        s0 = pltpu.async_copy(
                            gbuf0, acc.at[didx.at[j0]], sem2, add=True)
                        pltpu.make_async_copy(
                            y_core.at[sidx.at[j0 + 1]], gbuf1, sem1).wait()
                        s1 = pltpu.async_copy(
                            gbuf1, acc.at[didx.at[j0 + 1]], sem3, add=True)
                        s0.wait()

                        @pl.when(m < npair - 1)
                        def _():
                            pltpu.async_copy(
                                y_core.at[sidx.at[j0 + 2]], gbuf0, sem0)
                        s1.wait()

                        @pl.when(m < npair - 1)
                        def _():
                            pltpu.async_copy(
                                y_core.at[sidx.at[j0 + 3]], gbuf1, sem1)
                        return 0
                    lax.fori_loop(0, npair, pair, 0)

        def scatter_pass(y0, y1, out0, out1):
            pltpu.sync_copy(zeros_hbm, acc.at[pl.ds(row0, STRIPE)])
            if y0 is None:
                def fill(i, _):
                    for cc in range(W // 16):
                        gbuf0[i, pl.ds(cc * 16, 16)] = jnp.full(
                            (16,), 1.0, jnp.float32)
                    return 0
                lax.fori_loop(0, K, fill, 0)
            plsc.subcore_barrier()

            @pl.when(c == 0)
            def _():
                accumulate_edges(y0)
            if y1 is not None or out1 is not None:
                @pl.when(c == 1)
                def _():
                    accumulate_edges(y1)
            plsc.subcore_barrier()

            @pl.when(c == 0)
            def _():
                copyout(out0)
            if out1 is not None:
                @pl.when(c == 1)
                def _():
                    copyout(out1)

        for p in range(n_passes):
            scatter_pass(ys[NC * p], ys[NC * p + 1],
                         outs[NC * p], outs[NC * p + 1])
        if with_deg:
            scatter_pass(None, None, outs[n_slices], None)

    out_type = [jax.ShapeDtypeStruct((N_NODES, W), jnp.float32)] * n_out
    scratch = [
        pltpu.VMEM_SHARED((ACC_ROWS, W), jnp.float32),
        pltpu.VMEM((CHUNKS_H, K), jnp.int32),
        pltpu.VMEM((CHUNKS_H, K), jnp.int32),
        pltpu.VMEM((K, W), jnp.float32),
        pltpu.VMEM((K, W), jnp.float32),
        pltpu.SemaphoreType.DMA,
        pltpu.SemaphoreType.DMA,
        pltpu.SemaphoreType.DMA,
        pltpu.SemaphoreType.DMA,
    ]
    return pl.kernel(body, out_type=out_type, mesh=_MESH, scratch_types=scratch)


BN = 1000


def _layer_body(n_in, n_out, relu, *refs):
    aggs = refs[:n_in]
    ys = refs[n_in:2 * n_in]
    wl_ref, wr_ref, b_ref, deg_ref = refs[2 * n_in:2 * n_in + 4]
    o_refs = refs[2 * n_in + 4:]
    agg = jnp.concatenate([r[...] for r in aggs], axis=1)
    y = jnp.concatenate([r[...] for r in ys], axis=1)
    inv = 1.0 / jnp.maximum(deg_ref[:, 0:1], 1.0)
    acc = jnp.dot(agg * inv, wl_ref[...], preferred_element_type=jnp.float32)
    acc += jnp.dot(y, wr_ref[...], preferred_element_type=jnp.float32)
    acc += b_ref[...]
    if relu:
        acc = jnp.maximum(acc, 0.0)
    for i in range(n_out):
        o_refs[i][...] = acc[:, i * W:(i + 1) * W]


def _tc_layer(aggs, ys, wl, wr, b, deg, relu):
    n_in = len(aggs)
    din = n_in * W
    dout = wl.shape[1]
    n_out = dout // W
    sl = pl.BlockSpec((BN, W), lambda i: (i, 0))
    return pl.pallas_call(
        functools.partial(_layer_body, n_in, n_out, relu),
        grid=(N_NODES // BN,),
        in_specs=[sl] * (2 * n_in) + [
            pl.BlockSpec((din, dout), lambda i: (0, 0)),
            pl.BlockSpec((din, dout), lambda i: (0, 0)),
            pl.BlockSpec((1, dout), lambda i: (0, 0)),
            sl,
        ],
        out_specs=[sl] * n_out,
        out_shape=[jax.ShapeDtypeStruct((N_NODES, W), jnp.float32)] * n_out,
    )(*aggs, *ys, wl, wr, b.reshape(1, dout), deg)


def _mm_body(n_in, n_out, *refs):
    ys = refs[:n_in]
    w_ref, b_ref = refs[n_in:n_in + 2]
    o_refs = refs[n_in + 2:]
    y = jnp.concatenate([r[...] for r in ys], axis=1)
    acc = jnp.dot(y, w_ref[...], preferred_element_type=jnp.float32) + b_ref[...]
    if n_out == 1:
        o_refs[0][...] = acc
    else:
        for i in range(n_out):
            o_refs[i][...] = acc[:, i * W:(i + 1) * W]


def _tc_mm(ys, w, b, slice_out=True):
    n_in = len(ys)
    din = n_in * W
    dout = w.shape[1]
    n_out = dout // W if slice_out else 1
    ow = W if slice_out else dout
    sl = pl.BlockSpec((BN, W), lambda i: (i, 0))
    osl = pl.BlockSpec((BN, ow), lambda i: (i, 0))
    return pl.pallas_call(
        functools.partial(_mm_body, n_in, n_out),
        grid=(N_NODES // BN,),
        in_specs=[sl] * n_in + [
            pl.BlockSpec((din, dout), lambda i: (0, 0)),
            pl.BlockSpec((1, dout), lambda i: (0, 0)),
        ],
        out_specs=[osl] * n_out,
        out_shape=[jax.ShapeDtypeStruct((N_NODES, ow), jnp.float32)] * n_out,
    )(*ys, w, b.reshape(1, dout))


def _final_body(n_in, *refs):
    aggs = refs[:n_in]
    deg_ref, r_ref = refs[n_in:n_in + 2]
    o_ref = refs[n_in + 2]
    agg = jnp.concatenate([r[...] for r in aggs], axis=1)
    inv = 1.0 / jnp.maximum(deg_ref[:, 0:1], 1.0)
    o_ref[...] = agg * inv + r_ref[...]


def _tc_final(aggs, deg, r3):
    n_in = len(aggs)
    dout = n_in * W
    sl = pl.BlockSpec((BN, W), lambda i: (i, 0))
    return pl.pallas_call(
        functools.partial(_final_body, n_in),
        grid=(N_NODES // BN,),
        in_specs=[sl] * n_in + [sl, pl.BlockSpec((BN, dout), lambda i: (i, 0))],
        out_specs=pl.BlockSpec((BN, dout), lambda i: (i, 0)),
        out_shape=jax.ShapeDtypeStruct((N_NODES, dout), jnp.float32),
    )(*aggs, deg, r3)


def kernel(x, edge_index, Wl1, Wr1, b1, Wl2, Wr2, b2, Wl3, Wr3, b3):
    src = edge_index[0].astype(jnp.int32).reshape(NT, N_EDGES // NT)
    dst = edge_index[1].astype(jnp.int32).reshape(NT, N_EDGES // NT)
    src = jnp.concatenate([src, jnp.zeros((NT, PAD), jnp.int32)], axis=1)
    dst = jnp.concatenate(
        [dst, jnp.full((NT, PAD), N_NODES, jnp.int32)], axis=1)
    src = src.reshape(NT, HALVES, CHUNKS_H, K)
    dst = dst.reshape(NT, HALVES, CHUNKS_H, K)
    zeros = jnp.zeros((STRIPE, W), jnp.float32)

    xs = [lax.slice(x, (0, i * W), (N_NODES, (i + 1) * W)) for i in range(2)]

    agg1_0, agg1_1, deg = _make_agg(2, True)(src, dst, zeros, *xs)
    h1s = _tc_layer([agg1_0, agg1_1], xs, Wl1, Wr1, b1, deg, relu=True)

    agg2s = _make_agg(8, False)(src, dst, zeros, *h1s)
    h2s = _tc_layer(list(agg2s), list(h1s), Wl2, Wr2, b2, deg, relu=True)

    ts = _tc_mm(list(h2s), Wl3, jnp.zeros((512,), jnp.float32))
    agg3s = _make_agg(4, False)(src, dst, zeros, *ts)
    (r3,) = _tc_mm(list(h2s), Wr3, b3, slice_out=False)
    return _tc_final(list(agg3s), deg, r3)
